# baseline (speedup 1.0000x reference)
import functools

import jax
import jax.numpy as jnp
from jax import lax
from jax.experimental import pallas as pl
from jax.experimental.pallas import tpu as pltpu
from jax.experimental.pallas import tpu_sc as plsc

N = 10000
E = 320000
D = 128
HEADS = 4
CH = 32
NG = 16
DOUT = 64

NC, NS, LN = 2, 16, 16
NW = NC * NS
NPAD = 10240
DROW = 136
KE = 128

EG = E + N
KEG = 64
CPW_G = 164
CPG0, CPG1 = 196, 132
EPW_G = CPW_G * KEG
EGPAD = NW * EPW_G
CPW_S = 80
CPS0, CPS1 = 120, 40
EPW_S = CPW_S * KE
ESPAD = NW * EPW_S

RB = 1000
NBLK = N // RB

_mesh = plsc.VectorSubcoreMesh(core_axis_name="c", subcore_axis_name="s",
                               num_cores=NC, num_subcores=NS)


def _f32(*shape):
    return jax.ShapeDtypeStruct(shape, jnp.float32)


def _zero_and_dump(acc, zbuf, out_ref, cid, sid, drow, phase):
    base = sid * 640
    if phase == 0:
        def zb(r, _):
            def zc(c, _):
                zbuf[r, pl.ds(c * LN, LN)] = jnp.zeros((LN,), jnp.float32)
                return 0
            lax.fori_loop(0, drow // LN, zc, 0)
            return 0
        lax.fori_loop(0, LN, zb, 0)
        def zrow(j, _):
            pltpu.sync_copy(zbuf, acc.at[pl.ds(base + j * LN, LN)])
            return 0
        lax.fori_loop(0, 640 // LN, zrow, 0)
    else:
        pltpu.sync_copy(acc.at[pl.ds(base, 640)],
                        out_ref.at[cid, pl.ds(base, 640)])


_dnums = lax.GatherDimensionNumbers(
    offset_dims=(), collapsed_slice_dims=(0,), start_index_map=(0,))


def _reg_gather(v, idx):
    return lax.gather(v, idx[:, None], _dnums, (1,),
                      mode=lax.GatherScatterMode.PROMISE_IN_BOUNDS)


def _fetch_idx(pk_hbm, base, ke, pk_v, src_v, dst_v):
    pltpu.sync_copy(pk_hbm.at[pl.ds(base, ke)], pk_v)

    def seg(i, _):
        pk = pk_v[pl.ds(i * LN, LN)]
        src_v[pl.ds(i * LN, LN)] = pk >> 14
        dst_v[pl.ds(i * LN, LN)] = pk & 16383
        return 0
    lax.fori_loop(0, ke // LN, seg, 0)


def _gat_sc_body(t_hbm, at_hbm, gc_hbm, pk_hbm,
                 outf_hbm, outp_hbm,
                 pk0, sr0, sr1, sr2, sr3, ds0, ds1, ds2, ds3, in0, in1,
                 as0, as1, ad0, ad1, pc0, pc1, zbuf, zbuf2, gc_v, acc, accp,
                 sg0, sg1, sf0, sf1, sp0, sp1):
    cid = lax.axis_index("c")
    sid = lax.axis_index("s")

    _zero_and_dump(acc, zbuf, outf_hbm, cid, sid, D, 0)
    _zero_and_dump(accp, zbuf2, outp_hbm, cid, sid, LN, 0)
    pltpu.sync_copy(gc_hbm, gc_v)
    gc = gc_v[...]

    iot = jax.lax.broadcasted_iota(jnp.int32, (LN,), 0)
    shift4 = (iot & 3) + 4
    one16 = jnp.full((LN,), 1.0, jnp.float32)
    zero16 = jnp.zeros((LN,), jnp.float32)

    plsc.subcore_barrier()

    ins = (in0, in1)
    ass = (as0, as1)
    ads = (ad0, ad1)
    pcs = (pc0, pc1)
    srs = (sr0, sr1, sr2, sr3)
    dss = (ds0, ds1, ds2, ds3)
    sgs = (sg0, sg1)
    sfs = (sf0, sf1)
    sps = (sp0, sp1)

    def g_start(b, s):
        pltpu.async_copy(t_hbm.at[srs[s]], ins[b], sgs[b])
        pltpu.async_copy(at_hbm.at[srs[s]], ass[b], sgs[b])
        pltpu.async_copy(at_hbm.at[dss[s]], ads[b], sgs[b])

    def g_wait(b, s):
        pltpu.make_async_copy(t_hbm.at[srs[s]], ins[b], sgs[b]).wait()
        pltpu.make_async_copy(at_hbm.at[srs[s]], ass[b], sgs[b]).wait()
        pltpu.make_async_copy(at_hbm.at[dss[s]], ads[b], sgs[b]).wait()

    def s_start(b, s):
        pltpu.async_copy(ins[b], acc.at[dss[s]], sfs[b], add=True)
        pltpu.async_copy(pcs[b], accp.at[dss[s]], sps[b], add=True)

    def s_wait(b, s):
        pltpu.make_async_copy(ins[b], acc.at[dss[s]], sfs[b]).wait()
        pltpu.make_async_copy(pcs[b], accp.at[dss[s]], sps[b]).wait()

    def compute(b):
        in_buf = ins[b]
        as_buf = ass[b]
        ad_buf = ads[b]
        pc_buf = pcs[b]

        def row(r, _):
            a_s = as_buf[r, ...]
            a_d = _reg_gather(ad_buf[r, ...], shift4)
            asum = a_s + a_d
            alpha = jnp.where(asum > 0, asum, 0.2 * asum)
            gsum = gc + a_d
            m = jnp.where(gsum > 0, gsum, 0.2 * gsum)
            p = jnp.exp(alpha - m)
            pc_buf[r, ...] = jnp.where(
                iot < 4, p, jnp.where(iot == 4, one16, zero16))
            for h in range(HEADS):
                ph = _reg_gather(p, jnp.full((LN,), h, jnp.int32))
                for j in range(CH // LN):
                    cc = CH * h + LN * j
                    in_buf[r, pl.ds(cc, LN)] = in_buf[r, pl.ds(cc, LN)] * ph
            return 0
        lax.fori_loop(0, KEG, row, 0)

    nch = jnp.where(cid == 0, CPG0, CPG1)
    ebase = sid * (2 * EPW_G) + cid * (CPG0 * KEG)

    def fetch(c, s):
        _fetch_idx(pk_hbm, ebase + c * KEG, KEG, pk0, srs[s], dss[s])

    fetch(0, 0)
    fetch(1, 1)
    g_start(0, 0)

    def quad(u, _):
        c0 = u * 4
        for k in range(4):
            c = c0 + k
            d = k & 1

            @pl.when(c + 2 < nch)
            def _(c=c, k=k):
                fetch(c + 2, (k + 2) & 3)
            g_wait(d, k)
            if k == 0:
                @pl.when(u > 0)
                def _():
                    s_wait(1, 3)
            else:
                s_wait((k - 1) & 1, k - 1)

            @pl.when(c + 1 < nch)
            def _(d=d, k=k):
                g_start(1 - d, (k + 1) & 3)
            compute(d)
            s_start(d, k)
        return 0
    lax.fori_loop(0, nch // 4, quad, 0)
    s_wait(1, 3)

    plsc.subcore_barrier()
    _zero_and_dump(acc, zbuf, outf_hbm, cid, sid, D, 1)
    _zero_and_dump(accp, zbuf2, outp_hbm, cid, sid, LN, 1)


_sc_params = pltpu.CompilerParams(use_tc_tiling_on_sc=False,
                                  needs_layout_passes=False)

_gat_agg = pl.kernel(
    _gat_sc_body,
    out_type=(_f32(NC, NPAD, D), _f32(NC, NPAD, LN)),
    mesh=_mesh,
    compiler_params=_sc_params,
    scratch_types=[
        pltpu.VMEM((KEG,), jnp.int32),
        pltpu.VMEM((KEG,), jnp.int32),
        pltpu.VMEM((KEG,), jnp.int32),
        pltpu.VMEM((KEG,), jnp.int32),
        pltpu.VMEM((KEG,), jnp.int32),
        pltpu.VMEM((KEG,), jnp.int32),
        pltpu.VMEM((KEG,), jnp.int32),
        pltpu.VMEM((KEG,), jnp.int32),
        pltpu.VMEM((KEG,), jnp.int32),
        pltpu.VMEM((KEG, D), jnp.float32),
        pltpu.VMEM((KEG, D), jnp.float32),
        pltpu.VMEM((KEG, LN), jnp.float32),
        pltpu.VMEM((KEG, LN), jnp.float32),
        pltpu.VMEM((KEG, LN), jnp.float32),
        pltpu.VMEM((KEG, LN), jnp.float32),
        pltpu.VMEM((KEG, LN), jnp.float32),
        pltpu.VMEM((KEG, LN), jnp.float32),
        pltpu.VMEM((LN, D), jnp.float32),
        pltpu.VMEM((LN, LN), jnp.float32),
        pltpu.VMEM((LN,), jnp.float32),
        pltpu.VMEM_SHARED((NPAD, D), jnp.float32),
        pltpu.VMEM_SHARED((NPAD, LN), jnp.float32),
        pltpu.SemaphoreType.DMA,
        pltpu.SemaphoreType.DMA,
        pltpu.SemaphoreType.DMA,
        pltpu.SemaphoreType.DMA,
        pltpu.SemaphoreType.DMA,
        pltpu.SemaphoreType.DMA,
    ],
)


def _sage_sc_body(t_hbm, pk_hbm, out_hbm,
                  pk0, sr0, sr1, sr2, sr3, ds0, ds1, ds2, ds3, in0, in1,
                  zbuf, acc, sg0, sg1, ss0, ss1):
    cid = lax.axis_index("c")
    sid = lax.axis_index("s")

    _zero_and_dump(acc, zbuf, out_hbm, cid, sid, D, 0)
    plsc.subcore_barrier()

    ins = (in0, in1)
    srs = (sr0, sr1, sr2, sr3)
    dss = (ds0, ds1, ds2, ds3)
    sgs = (sg0, sg1)
    sss = (ss0, ss1)

    def g_start(b, s):
        pltpu.async_copy(t_hbm.at[srs[s]], ins[b], sgs[b])

    def g_wait(b, s):
        pltpu.make_async_copy(t_hbm.at[srs[s]], ins[b], sgs[b]).wait()

    def s_start(b, s):
        pltpu.async_copy(ins[b], acc.at[dss[s]], sss[b], add=True)

    def s_wait(b, s):
        pltpu.make_async_copy(ins[b], acc.at[dss[s]], sss[b]).wait()

    nch = jnp.where(cid == 0, CPS0, CPS1)
    ebase = sid * (2 * EPW_S) + cid * (CPS0 * KE)

    def fetch(c, s):
        _fetch_idx(pk_hbm, ebase + c * KE, KE, pk0, srs[s], dss[s])

    fetch(0, 0)
    fetch(1, 1)
    g_start(0, 0)

    def quad(u, _):
        c0 = u * 4
        for k in range(4):
            c = c0 + k
            d = k & 1

            @pl.when(c + 2 < nch)
            def _(c=c, k=k):
                fetch(c + 2, (k + 2) & 3)
            g_wait(d, k)
            if k == 0:
                @pl.when(u > 0)
                def _():
                    s_wait(1, 3)
            else:
                s_wait((k - 1) & 1, k - 1)

            @pl.when(c + 1 < nch)
            def _(d=d, k=k):
                g_start(1 - d, (k + 1) & 3)
            s_start(d, k)
        return 0
    lax.fori_loop(0, nch // 4, quad, 0)
    s_wait(1, 3)

    plsc.subcore_barrier()
    _zero_and_dump(acc, zbuf, out_hbm, cid, sid, D, 1)


_sage_agg = pl.kernel(
    _sage_sc_body,
    out_type=_f32(NC, NPAD, D),
    mesh=_mesh,
    compiler_params=_sc_params,
    scratch_types=[
        pltpu.VMEM((KE,), jnp.int32),
        pltpu.VMEM((KE,), jnp.int32),
        pltpu.VMEM((KE,), jnp.int32),
        pltpu.VMEM((KE,), jnp.int32),
        pltpu.VMEM((KE,), jnp.int32),
        pltpu.VMEM((KE,), jnp.int32),
        pltpu.VMEM((KE,), jnp.int32),
        pltpu.VMEM((KE,), jnp.int32),
        pltpu.VMEM((KE,), jnp.int32),
        pltpu.VMEM((KE, D), jnp.float32),
        pltpu.VMEM((KE, D), jnp.float32),
        pltpu.VMEM((LN, D), jnp.float32),
        pltpu.VMEM_SHARED((NPAD, D), jnp.float32),
        pltpu.SemaphoreType.DMA,
        pltpu.SemaphoreType.DMA,
        pltpu.SemaphoreType.DMA,
        pltpu.SemaphoreType.DMA,
    ],
)


_rowspec = pl.BlockSpec((RB, D), lambda i: (i, 0))
_fullw = pl.BlockSpec((D, D), lambda i: (0, 0))
_statspec = pl.BlockSpec((8, D), lambda i: (0, 0))


def _dot(a, b):
    return jax.lax.dot_general(a, b, (((1,), (0,)), ((), ())),
                               preferred_element_type=jnp.float32)


def _lrelu(v):
    return jnp.where(v > 0, v, 0.2 * v)


def _bn_apply(y, st_ref, g_ref, b_ref):
    mu = st_ref[0:1, :] / N
    var = st_ref[1:2, :] / N - mu * mu
    inv = jax.lax.rsqrt(var + 1e-5)
    return (y - mu) * inv * g_ref[0:1, :] + b_ref[0:1, :]


def _gat_prep_out(h, as_ref, ad_ref, gm_ref, t_ref, att_ref, i):
    asrc = _dot(h, as_ref[...])
    adst = _dot(h, ad_ref[...])
    t_ref[...] = h
    att_ref[...] = jnp.concatenate(
        [asrc, adst, jnp.zeros((RB, LN - 2 * HEADS), jnp.float32)], axis=1)
    prev = jnp.where(i == 0, jnp.full((1, HEADS), -1e30, jnp.float32),
                     gm_ref[0:1, 0:HEADS])
    gm_ref[0:1, 0:HEADS] = jnp.maximum(prev, jnp.max(asrc, axis=0, keepdims=True))


def _stage_a_body(x_ref, w_ref, as_ref, ad_ref, wr_ref, br_ref,
                  t_ref, adt_ref, res_ref, gm_ref):
    i = pl.program_id(0)
    xb = x_ref[...]
    h = _dot(xb, w_ref[...])
    res_ref[...] = _dot(xb, wr_ref[...]) + br_ref[0:1, :]
    _gat_prep_out(h, as_ref, ad_ref, gm_ref, t_ref, adt_ref, i)


_stage_a = pl.pallas_call(
    _stage_a_body,
    grid=(NBLK,),
    in_specs=[_rowspec, _fullw,
              pl.BlockSpec((D, HEADS), lambda i: (0, 0)),
              pl.BlockSpec((D, HEADS), lambda i: (0, 0)),
              _fullw, pl.BlockSpec((1, D), lambda i: (0, 0))],
    out_specs=[_rowspec,
               pl.BlockSpec((RB, LN), lambda i: (i, 0)),
               _rowspec, _statspec],
    out_shape=[_f32(N, D), _f32(N, LN), _f32(N, D), _f32(8, D)],
)


def _gat_combine_body(acc_ref, accp_ref, bg_ref, y_ref, st_ref, ic_ref):
    i = pl.program_id(0)
    s = acc_ref[0] + acc_ref[1]
    pc = accp_ref[0] + accp_ref[1]
    den = jnp.concatenate(
        [jnp.broadcast_to(pc[:, h:h + 1], (RB, CH)) for h in range(HEADS)],
        axis=1)
    y = s / (den + 1e-16) + bg_ref[0:1, :]
    y_ref[...] = y
    prev0 = jnp.where(i == 0, jnp.zeros((1, D)), st_ref[0:1, :])
    prev1 = jnp.where(i == 0, jnp.zeros((1, D)), st_ref[1:2, :])
    st_ref[0:1, :] = prev0 + jnp.sum(y, axis=0, keepdims=True)
    st_ref[1:2, :] = prev1 + jnp.sum(y * y, axis=0, keepdims=True)
    ic_ref[...] = 1.0 / jnp.maximum(pc[:, HEADS:HEADS + 1] - 1.0, 1.0)


_gat_combine = pl.pallas_call(
    _gat_combine_body,
    grid=(NBLK,),
    in_specs=[pl.BlockSpec((NC, RB, D), lambda i: (0, i, 0)),
              pl.BlockSpec((NC, RB, LN), lambda i: (0, i, 0)),
              pl.BlockSpec((1, D), lambda i: (0, 0))],
    out_specs=[_rowspec, _statspec, pl.BlockSpec((RB, 1), lambda i: (i, 0))],
    out_shape=[_f32(N, D), _f32(8, D), _f32(N, 1)],
)


def _gat_apply_prep_body(y_ref, st_ref, res_ref, g_ref, b_ref,
                         w_ref, as_ref, ad_ref, wr_ref, br_ref,
                         t_ref, adt_ref, res2_ref, gm_ref):
    i = pl.program_id(0)
    z = _lrelu(_bn_apply(y_ref[...], st_ref, g_ref, b_ref) + res_ref[...])
    h = _dot(z, w_ref[...])
    res2_ref[...] = _dot(z, wr_ref[...]) + br_ref[0:1, :]
    _gat_prep_out(h, as_ref, ad_ref, gm_ref, t_ref, adt_ref, i)


_gat_apply_prep = pl.pallas_call(
    _gat_apply_prep_body,
    grid=(NBLK,),
    in_specs=[_rowspec, _statspec, _rowspec,
              pl.BlockSpec((1, D), lambda i: (0, 0)),
              pl.BlockSpec((1, D), lambda i: (0, 0)),
              _fullw,
              pl.BlockSpec((D, HEADS), lambda i: (0, 0)),
              pl.BlockSpec((D, HEADS), lambda i: (0, 0)),
              _fullw, pl.BlockSpec((1, D), lambda i: (0, 0))],
    out_specs=[_rowspec,
               pl.BlockSpec((RB, LN), lambda i: (i, 0)),
               _rowspec, _statspec],
    out_shape=[_f32(N, D), _f32(N, LN), _f32(N, D), _f32(8, D)],
)


def _gat_apply_sage_body(y_ref, st_ref, res_ref, g_ref, b_ref, wr_ref,
                         z_ref, r_ref):
    z = _lrelu(_bn_apply(y_ref[...], st_ref, g_ref, b_ref) + res_ref[...])
    z_ref[...] = z
    r_ref[...] = _dot(z, wr_ref[...])


_gat_apply_sage = pl.pallas_call(
    _gat_apply_sage_body,
    grid=(NBLK,),
    in_specs=[_rowspec, _statspec, _rowspec,
              pl.BlockSpec((1, D), lambda i: (0, 0)),
              pl.BlockSpec((1, D), lambda i: (0, 0)),
              _fullw],
    out_specs=[_rowspec, _rowspec],
    out_shape=[_f32(N, D), _f32(N, D)],
)


def _sage_combine_body(acc_ref, ic_ref, r_ref, wl_ref, bl_ref, y_ref, st_ref):
    i = pl.program_id(0)
    s = acc_ref[0] + acc_ref[1]
    mean = s * ic_ref[...]
    y = _dot(mean, wl_ref[...]) + bl_ref[0:1, :] + r_ref[...]
    y_ref[...] = y
    prev0 = jnp.where(i == 0, jnp.zeros((1, D)), st_ref[0:1, :])
    prev1 = jnp.where(i == 0, jnp.zeros((1, D)), st_ref[1:2, :])
    st_ref[0:1, :] = prev0 + jnp.sum(y, axis=0, keepdims=True)
    st_ref[1:2, :] = prev1 + jnp.sum(y * y, axis=0, keepdims=True)


_sage_combine = pl.pallas_call(
    _sage_combine_body,
    grid=(NBLK,),
    in_specs=[pl.BlockSpec((NC, RB, D), lambda i: (0, i, 0)),
              pl.BlockSpec((RB, 1), lambda i: (i, 0)),
              _rowspec, _fullw, pl.BlockSpec((1, D), lambda i: (0, 0))],
    out_specs=[_rowspec, _statspec],
    out_shape=[_f32(N, D), _f32(8, D)],
)


def _sage_apply_body(y_ref, st_ref, g_ref, b_ref, wr_ref, z_ref, r_ref):
    z = _lrelu(_bn_apply(y_ref[...], st_ref, g_ref, b_ref))
    z_ref[...] = z
    r_ref[...] = _dot(z, wr_ref[...])


_sage_apply = pl.pallas_call(
    _sage_apply_body,
    grid=(NBLK,),
    in_specs=[_rowspec, _statspec,
              pl.BlockSpec((1, D), lambda i: (0, 0)),
              pl.BlockSpec((1, D), lambda i: (0, 0)),
              _fullw],
    out_specs=[_rowspec, _rowspec],
    out_shape=[_f32(N, D), _f32(N, D)],
)


def _final_body(y_ref, st_ref, g_ref, b_ref, bt_ref, wf_ref, bf_ref,
                out_ref, gacc):
    i = pl.program_id(0)
    z = _lrelu(_bn_apply(y_ref[...], st_ref, g_ref, b_ref))
    zext = jnp.concatenate([z, jnp.ones((RB, 8), jnp.float32)], axis=1)
    gid = jax.lax.broadcasted_iota(jnp.int32, (NG, 1), 0).astype(jnp.float32)
    oh = jnp.where(gid == bt_ref[0], 1.0, 0.0)
    prev = jnp.where(i == 0, jnp.zeros((NG, D + 8)), gacc[...])
    gsum = prev + _dot(oh, zext)
    gacc[...] = gsum

    @pl.when(i == NBLK - 1)
    def _():
        gm = gsum[:, :D] / jnp.maximum(gsum[:, D:D + 1], 1.0)
        out_ref[...] = _dot(gm, wf_ref[...]) + bf_ref[0:1, :]


_final = pl.pallas_call(
    _final_body,
    grid=(NBLK,),
    in_specs=[_rowspec, _statspec,
              pl.BlockSpec((1, D), lambda i: (0, 0)),
              pl.BlockSpec((1, D), lambda i: (0, 0)),
              pl.BlockSpec((1, 1, RB), lambda i: (i, 0, 0)),
              pl.BlockSpec((D, DOUT), lambda i: (0, 0)),
              pl.BlockSpec((1, DOUT), lambda i: (0, 0))],
    out_specs=pl.BlockSpec((NG, DOUT), lambda i: (0, 0)),
    out_shape=_f32(NG, DOUT),
    scratch_shapes=[pltpu.VMEM((NG, D + 8), jnp.float32)],
)


def _att_mat(att):
    m = jnp.zeros((D, HEADS), jnp.float32)
    for h in range(HEADS):
        m = m.at[h * CH:(h + 1) * CH, h].set(att[0, h])
    return m


def _row(v):
    return v.reshape(1, -1)


def kernel(x, edge_index, batch, params):
    p = params
    loop = jnp.arange(N, dtype=jnp.int32)
    padg = EGPAD - EG
    gpk = jnp.concatenate([
        (edge_index[0] << 14) | edge_index[1],
        (loop << 14) | loop,
        jnp.full((padg,), N, jnp.int32)])
    pads = ESPAD - E
    spk = jnp.concatenate([
        (edge_index[0] << 14) | edge_index[1],
        jnp.full((pads,), N, jnp.int32)])
    batf = batch.astype(jnp.float32).reshape(NBLK, 1, RB)

    g1 = p['gat1']
    t1, att1, res1, gm1 = _stage_a(
        x, g1['W'], _att_mat(g1['att_src']), _att_mat(g1['att_dst']),
        p['res1']['W'], _row(p['res1']['b']))
    gc1 = jnp.tile(gm1[0, :HEADS], HEADS)
    att1 = jnp.concatenate([att1, jnp.zeros((LN, LN), jnp.float32)])
    acc1, accp1 = _gat_agg(t1, att1, gc1, gpk)
    y1, st1, invc = _gat_combine(acc1, accp1, _row(g1['b']))

    g2 = p['gat2']
    t2, att2, res2, gm2 = _gat_apply_prep(
        y1, st1, res1, _row(p['bn1']['g']), _row(p['bn1']['b']),
        g2['W'], _att_mat(g2['att_src']), _att_mat(g2['att_dst']),
        p['res2']['W'], _row(p['res2']['b']))
    gc2 = jnp.tile(gm2[0, :HEADS], HEADS)
    att2 = jnp.concatenate([att2, jnp.zeros((LN, LN), jnp.float32)])
    acc2, accp2 = _gat_agg(t2, att2, gc2, gpk)
    y2, st2, _ = _gat_combine(acc2, accp2, _row(g2['b']))

    z2, r3 = _gat_apply_sage(y2, st2, res2, _row(p['bn2']['g']),
                             _row(p['bn2']['b']), p['sage3']['Wr'])
    acc3 = _sage_agg(z2, spk)
    y3, st3 = _sage_combine(acc3, invc, r3, p['sage3']['Wl'],
                            _row(p['sage3']['bl']))

    z3, r4 = _sage_apply(y3, st3, _row(p['bn3']['g']), _row(p['bn3']['b']),
                         p['sage4']['Wr'])
    acc4 = _sage_agg(z3, spk)
    y4, st4 = _sage_combine(acc4, invc, r4, p['sage4']['Wl'],
                            _row(p['sage4']['bl']))

    z4, r5 = _sage_apply(y4, st4, _row(p['bn4']['g']), _row(p['bn4']['b']),
                         p['sage5']['Wr'])
    acc5 = _sage_agg(z4, spk)
    y5, st5 = _sage_combine(acc5, invc, r5, p['sage5']['Wl'],
                            _row(p['sage5']['bl']))

    return _final(y5, st5, _row(p['bn5']['g']), _row(p['bn5']['b']),
                  batf, p['fc']['W'], _row(p['fc']['b']))


# speedup vs baseline: 1.0130x; 1.0130x over previous
import jax
import jax.numpy as jnp
from jax import lax
from jax.experimental import pallas as pl
from jax.experimental.pallas import tpu as pltpu
from jax.experimental.pallas import tpu_sc as plsc

N = 10000
E = 320000
D = 128
HEADS = 4
CH = 32
NG = 16
DOUT = 64

NC, NS, LN = 2, 16, 16
NW = NC * NS
NPAD = 10240
KE = 128

EG = E + N
KEG = 64
CPW_G = 164
CPG0, CPG1 = 184, 144
EPW_G = CPW_G * KEG
EGPAD = NW * EPW_G
CPW_S = 80
CPS0, CPS1 = 120, 40
EPW_S = CPW_S * KE
ESPAD = NW * EPW_S

RB = 1000
NBLK = N // RB

_mesh = plsc.VectorSubcoreMesh(core_axis_name="c", subcore_axis_name="s",
                               num_cores=NC, num_subcores=NS)


def _f32(*shape):
    return jax.ShapeDtypeStruct(shape, jnp.float32)


def _zero_and_dump(acc, zbuf, out_ref, cid, sid, drow, phase):
    base = sid * 640
    if phase == 0:
        def zb(r, _):
            def zc(c, _):
                zbuf[r, pl.ds(c * LN, LN)] = jnp.zeros((LN,), jnp.float32)
                return 0
            lax.fori_loop(0, drow // LN, zc, 0)
            return 0
        lax.fori_loop(0, LN, zb, 0)
        def zrow(j, _):
            pltpu.sync_copy(zbuf, acc.at[pl.ds(base + j * LN, LN)])
            return 0
        lax.fori_loop(0, 640 // LN, zrow, 0)
    else:
        pltpu.sync_copy(acc.at[pl.ds(base, 640)],
                        out_ref.at[cid, pl.ds(base, 640)])


_dnums = lax.GatherDimensionNumbers(
    offset_dims=(), collapsed_slice_dims=(0,), start_index_map=(0,))


def _reg_gather(v, idx):
    return lax.gather(v, idx[:, None], _dnums, (1,),
                      mode=lax.GatherScatterMode.PROMISE_IN_BOUNDS)


def _fetch_idx(pk_hbm, base, ke, pk_v, src_v, dst_v):
    pltpu.sync_copy(pk_hbm.at[pl.ds(base, ke)], pk_v)

    def seg(i, _):
        pk = pk_v[pl.ds(i * LN, LN)]
        src_v[pl.ds(i * LN, LN)] = pk >> 14
        dst_v[pl.ds(i * LN, LN)] = pk & 16383
        return 0
    lax.fori_loop(0, ke // LN, seg, 0)


def _gat_sc_body(t_hbm, at_hbm, gc_hbm, pk_hbm,
                 outf_hbm, outp_hbm,
                 pk0, pk1, srd0, srd1, dsd0, dsd1, in0, in1,
                 as0, as1, ad0, ad1, pc0, pc1, zbuf, zbuf2, gc_v, acc, accp,
                 sg0, sg1, sf0, sf1, sp0, sp1):
    cid = lax.axis_index("c")
    sid = lax.axis_index("s")
    npair = jnp.where(cid == 0, CPG0 // 2, CPG1 // 2)

    _zero_and_dump(acc, zbuf, outf_hbm, cid, sid, D, 0)
    _zero_and_dump(accp, zbuf2, outp_hbm, cid, sid, LN, 0)
    pltpu.sync_copy(gc_hbm, gc_v)
    gc = gc_v[...]

    iot = jax.lax.broadcasted_iota(jnp.int32, (LN,), 0)
    shift4 = (iot & 3) + 4
    one16 = jnp.full((LN,), 1.0, jnp.float32)
    zero16 = jnp.zeros((LN,), jnp.float32)

    plsc.subcore_barrier()

    pks = (pk0, pk1)
    srs = (srd0, srd1)
    dss = (dsd0, dsd1)
    ins = (in0, in1)
    ass = (as0, as1)
    ads = (ad0, ad1)
    pcs = (pc0, pc1)
    sgs = (sg0, sg1)
    sfs = (sf0, sf1)
    sps = (sp0, sp1)

    def g_start(c, b):
        pltpu.async_copy(t_hbm.at[srs[b]], ins[b], sgs[b])
        pltpu.async_copy(at_hbm.at[srs[b]], ass[b], sgs[b])
        pltpu.async_copy(at_hbm.at[dss[b]], ads[b], sgs[b])

    def g_wait(c, b):
        pltpu.make_async_copy(t_hbm.at[srs[b]], ins[b], sgs[b]).wait()
        pltpu.make_async_copy(at_hbm.at[srs[b]], ass[b], sgs[b]).wait()
        pltpu.make_async_copy(at_hbm.at[dss[b]], ads[b], sgs[b]).wait()

    def s_start(c, b):
        pltpu.async_copy(ins[b], acc.at[dss[b]], sfs[b], add=True)
        pltpu.async_copy(pcs[b], accp.at[dss[b]], sps[b], add=True)

    def s_wait(c, b):
        pltpu.make_async_copy(ins[b], acc.at[dss[b]], sfs[b]).wait()
        pltpu.make_async_copy(pcs[b], accp.at[dss[b]], sps[b]).wait()

    def compute(b):
        in_buf = ins[b]
        as_buf = ass[b]
        ad_buf = ads[b]
        pc_buf = pcs[b]

        def row(r, _):
            a_s = as_buf[r, ...]
            a_d = _reg_gather(ad_buf[r, ...], shift4)
            asum = a_s + a_d
            alpha = jnp.where(asum > 0, asum, 0.2 * asum)
            gsum = gc + a_d
            m = jnp.where(gsum > 0, gsum, 0.2 * gsum)
            p = jnp.exp(alpha - m)
            pc_buf[r, ...] = jnp.where(
                iot < 4, p, jnp.where(iot == 4, one16, zero16))
            for h in range(HEADS):
                ph = _reg_gather(p, jnp.full((LN,), h, jnp.int32))
                for j in range(CH // LN):
                    cc = CH * h + LN * j
                    in_buf[r, pl.ds(cc, LN)] = in_buf[r, pl.ds(cc, LN)] * ph
            return 0
        lax.fori_loop(0, KEG, row, 0)

    ebase = sid * (2 * EPW_G) + cid * (CPG0 * KEG)
    _fetch_idx(pk_hbm, ebase, KEG, pk0, srd0, dsd0)
    g_start(0, 0)

    def pair(q, _):
        c0 = q * 2
        c1 = c0 + 1

        @pl.when(q > 0)
        def _():
            s_wait(c1 - 2, 1)
        _fetch_idx(pk_hbm, ebase + c1 * KEG, KEG, pk1, srd1, dsd1)
        g_start(c1, 1)
        g_wait(c0, 0)
        compute(0)
        s_start(c0, 0)

        s_wait(c0, 0)

        @pl.when(q < npair - 1)
        def _():
            _fetch_idx(pk_hbm, ebase + (c0 + 2) * KEG, KEG, pk0, srd0, dsd0)
            g_start(c0 + 2, 0)
        g_wait(c1, 1)
        compute(1)
        s_start(c1, 1)
        return 0
    lax.fori_loop(0, npair, pair, 0)
    s_wait(0, 1)

    plsc.subcore_barrier()
    _zero_and_dump(acc, zbuf, outf_hbm, cid, sid, D, 1)
    _zero_and_dump(accp, zbuf2, outp_hbm, cid, sid, LN, 1)


_sc_params = pltpu.CompilerParams(use_tc_tiling_on_sc=False,
                                  needs_layout_passes=False)

_gat_agg = pl.kernel(
    _gat_sc_body,
    out_type=(_f32(NC, NPAD, D), _f32(NC, NPAD, LN)),
    mesh=_mesh,
    compiler_params=_sc_params,
    scratch_types=[
        pltpu.VMEM((KEG,), jnp.int32),
        pltpu.VMEM((KEG,), jnp.int32),
        pltpu.VMEM((KEG,), jnp.int32),
        pltpu.VMEM((KEG,), jnp.int32),
        pltpu.VMEM((KEG,), jnp.int32),
        pltpu.VMEM((KEG,), jnp.int32),
        pltpu.VMEM((KEG, D), jnp.float32),
        pltpu.VMEM((KEG, D), jnp.float32),
        pltpu.VMEM((KEG, LN), jnp.float32),
        pltpu.VMEM((KEG, LN), jnp.float32),
        pltpu.VMEM((KEG, LN), jnp.float32),
        pltpu.VMEM((KEG, LN), jnp.float32),
        pltpu.VMEM((KEG, LN), jnp.float32),
        pltpu.VMEM((KEG, LN), jnp.float32),
        pltpu.VMEM((LN, D), jnp.float32),
        pltpu.VMEM((LN, LN), jnp.float32),
        pltpu.VMEM((LN,), jnp.float32),
        pltpu.VMEM_SHARED((NPAD, D), jnp.float32),
        pltpu.VMEM_SHARED((NPAD, LN), jnp.float32),
        pltpu.SemaphoreType.DMA,
        pltpu.SemaphoreType.DMA,
        pltpu.SemaphoreType.DMA,
        pltpu.SemaphoreType.DMA,
        pltpu.SemaphoreType.DMA,
        pltpu.SemaphoreType.DMA,
    ],
)


def _sage_sc_body(t_hbm, pk_hbm, out_hbm,
                  pk0, pk1, srd0, srd1, dsd0, dsd1, in0, in1, zbuf, acc,
                  sg0, sg1, ss0, ss1):
    cid = lax.axis_index("c")
    sid = lax.axis_index("s")
    npair = jnp.where(cid == 0, CPS0 // 2, CPS1 // 2)

    _zero_and_dump(acc, zbuf, out_hbm, cid, sid, D, 0)
    plsc.subcore_barrier()

    pks = (pk0, pk1)
    srs = (srd0, srd1)
    dss = (dsd0, dsd1)
    ins = (in0, in1)
    sgs = (sg0, sg1)
    sss = (ss0, ss1)

    def g_start(b):
        pltpu.async_copy(t_hbm.at[srs[b]], ins[b], sgs[b])

    def g_wait(b):
        pltpu.make_async_copy(t_hbm.at[srs[b]], ins[b], sgs[b]).wait()

    def s_start(b):
        pltpu.async_copy(ins[b], acc.at[dss[b]], sss[b], add=True)

    def s_wait(b):
        pltpu.make_async_copy(ins[b], acc.at[dss[b]], sss[b]).wait()

    ebase = sid * (2 * EPW_S) + cid * (CPS0 * KE)
    _fetch_idx(pk_hbm, ebase, KE, pk0, srd0, dsd0)
    g_start(0)

    def pair(q, _):
        c0 = q * 2
        c1 = c0 + 1

        @pl.when(q > 0)
        def _():
            s_wait(1)
        _fetch_idx(pk_hbm, ebase + c1 * KE, KE, pk1, srd1, dsd1)
        g_start(1)
        g_wait(0)
        s_start(0)

        s_wait(0)

        @pl.when(q < npair - 1)
        def _():
            _fetch_idx(pk_hbm, ebase + (c0 + 2) * KE, KE, pk0, srd0, dsd0)
            g_start(0)
        g_wait(1)
        s_start(1)
        return 0
    lax.fori_loop(0, npair, pair, 0)
    s_wait(1)

    plsc.subcore_barrier()
    _zero_and_dump(acc, zbuf, out_hbm, cid, sid, D, 1)


_sage_agg = pl.kernel(
    _sage_sc_body,
    out_type=_f32(NC, NPAD, D),
    mesh=_mesh,
    compiler_params=_sc_params,
    scratch_types=[
        pltpu.VMEM((KE,), jnp.int32),
        pltpu.VMEM((KE,), jnp.int32),
        pltpu.VMEM((KE,), jnp.int32),
        pltpu.VMEM((KE,), jnp.int32),
        pltpu.VMEM((KE,), jnp.int32),
        pltpu.VMEM((KE,), jnp.int32),
        pltpu.VMEM((KE, D), jnp.float32),
        pltpu.VMEM((KE, D), jnp.float32),
        pltpu.VMEM((LN, D), jnp.float32),
        pltpu.VMEM_SHARED((NPAD, D), jnp.float32),
        pltpu.SemaphoreType.DMA,
        pltpu.SemaphoreType.DMA,
        pltpu.SemaphoreType.DMA,
        pltpu.SemaphoreType.DMA,
    ],
)


_rowspec = pl.BlockSpec((RB, D), lambda i: (i, 0))
_fullw = pl.BlockSpec((D, D), lambda i: (0, 0))
_statspec = pl.BlockSpec((8, D), lambda i: (0, 0))


def _dot(a, b):
    return jax.lax.dot_general(a, b, (((1,), (0,)), ((), ())),
                               preferred_element_type=jnp.float32)


def _lrelu(v):
    return jnp.where(v > 0, v, 0.2 * v)


def _bn_apply(y, st_ref, g_ref, b_ref):
    mu = st_ref[0:1, :] / N
    var = st_ref[1:2, :] / N - mu * mu
    inv = jax.lax.rsqrt(var + 1e-5)
    return (y - mu) * inv * g_ref[0:1, :] + b_ref[0:1, :]


def _gat_prep_out(h, as_ref, ad_ref, gm_ref, t_ref, att_ref, i):
    asrc = _dot(h, as_ref[...])
    adst = _dot(h, ad_ref[...])
    t_ref[...] = h
    att_ref[...] = jnp.concatenate(
        [asrc, adst, jnp.zeros((RB, LN - 2 * HEADS), jnp.float32)], axis=1)
    prev = jnp.where(i == 0, jnp.full((1, HEADS), -1e30, jnp.float32),
                     gm_ref[0:1, 0:HEADS])
    gm_ref[0:1, 0:HEADS] = jnp.maximum(prev, jnp.max(asrc, axis=0, keepdims=True))


def _stage_a_body(x_ref, w_ref, as_ref, ad_ref, wr_ref, br_ref,
                  t_ref, adt_ref, res_ref, gm_ref):
    i = pl.program_id(0)
    xb = x_ref[...]
    h = _dot(xb, w_ref[...])
    res_ref[...] = _dot(xb, wr_ref[...]) + br_ref[0:1, :]
    _gat_prep_out(h, as_ref, ad_ref, gm_ref, t_ref, adt_ref, i)


_stage_a = pl.pallas_call(
    _stage_a_body,
    grid=(NBLK,),
    in_specs=[_rowspec, _fullw,
              pl.BlockSpec((D, HEADS), lambda i: (0, 0)),
              pl.BlockSpec((D, HEADS), lambda i: (0, 0)),
              _fullw, pl.BlockSpec((1, D), lambda i: (0, 0))],
    out_specs=[_rowspec,
               pl.BlockSpec((RB, LN), lambda i: (i, 0)),
               _rowspec, _statspec],
    out_shape=[_f32(N, D), _f32(N, LN), _f32(N, D), _f32(8, D)],
)


def _gat_combine_body(acc_ref, accp_ref, bg_ref, y_ref, st_ref, ic_ref):
    i = pl.program_id(0)
    s = acc_ref[0] + acc_ref[1]
    pc = accp_ref[0] + accp_ref[1]
    den = jnp.concatenate(
        [jnp.broadcast_to(pc[:, h:h + 1], (RB, CH)) for h in range(HEADS)],
        axis=1)
    y = s / (den + 1e-16) + bg_ref[0:1, :]
    y_ref[...] = y
    prev0 = jnp.where(i == 0, jnp.zeros((1, D)), st_ref[0:1, :])
    prev1 = jnp.where(i == 0, jnp.zeros((1, D)), st_ref[1:2, :])
    st_ref[0:1, :] = prev0 + jnp.sum(y, axis=0, keepdims=True)
    st_ref[1:2, :] = prev1 + jnp.sum(y * y, axis=0, keepdims=True)
    ic_ref[...] = 1.0 / jnp.maximum(pc[:, HEADS:HEADS + 1] - 1.0, 1.0)


_gat_combine = pl.pallas_call(
    _gat_combine_body,
    grid=(NBLK,),
    in_specs=[pl.BlockSpec((NC, RB, D), lambda i: (0, i, 0)),
              pl.BlockSpec((NC, RB, LN), lambda i: (0, i, 0)),
              pl.BlockSpec((1, D), lambda i: (0, 0))],
    out_specs=[_rowspec, _statspec, pl.BlockSpec((RB, 1), lambda i: (i, 0))],
    out_shape=[_f32(N, D), _f32(8, D), _f32(N, 1)],
)


def _gat_apply_prep_body(y_ref, st_ref, res_ref, g_ref, b_ref,
                         w_ref, as_ref, ad_ref, wr_ref, br_ref,
                         t_ref, adt_ref, res2_ref, gm_ref):
    i = pl.program_id(0)
    z = _lrelu(_bn_apply(y_ref[...], st_ref, g_ref, b_ref) + res_ref[...])
    h = _dot(z, w_ref[...])
    res2_ref[...] = _dot(z, wr_ref[...]) + br_ref[0:1, :]
    _gat_prep_out(h, as_ref, ad_ref, gm_ref, t_ref, adt_ref, i)


_gat_apply_prep = pl.pallas_call(
    _gat_apply_prep_body,
    grid=(NBLK,),
    in_specs=[_rowspec, _statspec, _rowspec,
              pl.BlockSpec((1, D), lambda i: (0, 0)),
              pl.BlockSpec((1, D), lambda i: (0, 0)),
              _fullw,
              pl.BlockSpec((D, HEADS), lambda i: (0, 0)),
              pl.BlockSpec((D, HEADS), lambda i: (0, 0)),
              _fullw, pl.BlockSpec((1, D), lambda i: (0, 0))],
    out_specs=[_rowspec,
               pl.BlockSpec((RB, LN), lambda i: (i, 0)),
               _rowspec, _statspec],
    out_shape=[_f32(N, D), _f32(N, LN), _f32(N, D), _f32(8, D)],
)


def _gat_apply_sage_body(y_ref, st_ref, res_ref, g_ref, b_ref, wr_ref,
                         z_ref, r_ref):
    z = _lrelu(_bn_apply(y_ref[...], st_ref, g_ref, b_ref) + res_ref[...])
    z_ref[...] = z
    r_ref[...] = _dot(z, wr_ref[...])


_gat_apply_sage = pl.pallas_call(
    _gat_apply_sage_body,
    grid=(NBLK,),
    in_specs=[_rowspec, _statspec, _rowspec,
              pl.BlockSpec((1, D), lambda i: (0, 0)),
              pl.BlockSpec((1, D), lambda i: (0, 0)),
              _fullw],
    out_specs=[_rowspec, _rowspec],
    out_shape=[_f32(N, D), _f32(N, D)],
)


def _sage_combine_body(acc_ref, ic_ref, r_ref, wl_ref, bl_ref, y_ref, st_ref):
    i = pl.program_id(0)
    s = acc_ref[0] + acc_ref[1]
    mean = s * ic_ref[...]
    y = _dot(mean, wl_ref[...]) + bl_ref[0:1, :] + r_ref[...]
    y_ref[...] = y
    prev0 = jnp.where(i == 0, jnp.zeros((1, D)), st_ref[0:1, :])
    prev1 = jnp.where(i == 0, jnp.zeros((1, D)), st_ref[1:2, :])
    st_ref[0:1, :] = prev0 + jnp.sum(y, axis=0, keepdims=True)
    st_ref[1:2, :] = prev1 + jnp.sum(y * y, axis=0, keepdims=True)


_sage_combine = pl.pallas_call(
    _sage_combine_body,
    grid=(NBLK,),
    in_specs=[pl.BlockSpec((NC, RB, D), lambda i: (0, i, 0)),
              pl.BlockSpec((RB, 1), lambda i: (i, 0)),
              _rowspec, _fullw, pl.BlockSpec((1, D), lambda i: (0, 0))],
    out_specs=[_rowspec, _statspec],
    out_shape=[_f32(N, D), _f32(8, D)],
)


def _sage_apply_body(y_ref, st_ref, g_ref, b_ref, wr_ref, z_ref, r_ref):
    z = _lrelu(_bn_apply(y_ref[...], st_ref, g_ref, b_ref))
    z_ref[...] = z
    r_ref[...] = _dot(z, wr_ref[...])


_sage_apply = pl.pallas_call(
    _sage_apply_body,
    grid=(NBLK,),
    in_specs=[_rowspec, _statspec,
              pl.BlockSpec((1, D), lambda i: (0, 0)),
              pl.BlockSpec((1, D), lambda i: (0, 0)),
              _fullw],
    out_specs=[_rowspec, _rowspec],
    out_shape=[_f32(N, D), _f32(N, D)],
)


def _final_body(y_ref, st_ref, g_ref, b_ref, bt_ref, wf_ref, bf_ref,
                out_ref, gacc):
    i = pl.program_id(0)
    z = _lrelu(_bn_apply(y_ref[...], st_ref, g_ref, b_ref))
    zext = jnp.concatenate([z, jnp.ones((RB, 8), jnp.float32)], axis=1)
    gid = jax.lax.broadcasted_iota(jnp.int32, (NG, 1), 0).astype(jnp.float32)
    oh = jnp.where(gid == bt_ref[0], 1.0, 0.0)
    prev = jnp.where(i == 0, jnp.zeros((NG, D + 8)), gacc[...])
    gsum = prev + _dot(oh, zext)
    gacc[...] = gsum

    @pl.when(i == NBLK - 1)
    def _():
        gm = gsum[:, :D] / jnp.maximum(gsum[:, D:D + 1], 1.0)
        out_ref[...] = _dot(gm, wf_ref[...]) + bf_ref[0:1, :]


_final = pl.pallas_call(
    _final_body,
    grid=(NBLK,),
    in_specs=[_rowspec, _statspec,
              pl.BlockSpec((1, D), lambda i: (0, 0)),
              pl.BlockSpec((1, D), lambda i: (0, 0)),
              pl.BlockSpec((1, 1, RB), lambda i: (i, 0, 0)),
              pl.BlockSpec((D, DOUT), lambda i: (0, 0)),
              pl.BlockSpec((1, DOUT), lambda i: (0, 0))],
    out_specs=pl.BlockSpec((NG, DOUT), lambda i: (0, 0)),
    out_shape=_f32(NG, DOUT),
    scratch_shapes=[pltpu.VMEM((NG, D + 8), jnp.float32)],
)


def _att_mat(att):
    m = jnp.zeros((D, HEADS), jnp.float32)
    for h in range(HEADS):
        m = m.at[h * CH:(h + 1) * CH, h].set(att[0, h])
    return m


def _row(v):
    return v.reshape(1, -1)


def kernel(x, edge_index, batch, params):
    p = params
    loop = jnp.arange(N, dtype=jnp.int32)
    padg = EGPAD - EG
    gpk = jnp.concatenate([
        (edge_index[0] << 14) | edge_index[1],
        (loop << 14) | loop,
        jnp.full((padg,), N, jnp.int32)])
    pads = ESPAD - E
    spk = jnp.concatenate([
        (edge_index[0] << 14) | edge_index[1],
        jnp.full((pads,), N, jnp.int32)])
    batf = batch.astype(jnp.float32).reshape(NBLK, 1, RB)

    g1 = p['gat1']
    t1, att1, res1, gm1 = _stage_a(
        x, g1['W'], _att_mat(g1['att_src']), _att_mat(g1['att_dst']),
        p['res1']['W'], _row(p['res1']['b']))
    gc1 = jnp.tile(gm1[0, :HEADS], HEADS)
    att1 = jnp.concatenate([att1, jnp.zeros((LN, LN), jnp.float32)])
    acc1, accp1 = _gat_agg(t1, att1, gc1, gpk)
    y1, st1, invc = _gat_combine(acc1, accp1, _row(g1['b']))

    g2 = p['gat2']
    t2, att2, res2, gm2 = _gat_apply_prep(
        y1, st1, res1, _row(p['bn1']['g']), _row(p['bn1']['b']),
        g2['W'], _att_mat(g2['att_src']), _att_mat(g2['att_dst']),
        p['res2']['W'], _row(p['res2']['b']))
    gc2 = jnp.tile(gm2[0, :HEADS], HEADS)
    att2 = jnp.concatenate([att2, jnp.zeros((LN, LN), jnp.float32)])
    acc2, accp2 = _gat_agg(t2, att2, gc2, gpk)
    y2, st2, _ = _gat_combine(acc2, accp2, _row(g2['b']))

    z2, r3 = _gat_apply_sage(y2, st2, res2, _row(p['bn2']['g']),
                             _row(p['bn2']['b']), p['sage3']['Wr'])
    acc3 = _sage_agg(z2, spk)
    y3, st3 = _sage_combine(acc3, invc, r3, p['sage3']['Wl'],
                            _row(p['sage3']['bl']))

    z3, r4 = _sage_apply(y3, st3, _row(p['bn3']['g']), _row(p['bn3']['b']),
                         p['sage4']['Wr'])
    acc4 = _sage_agg(z3, spk)
    y4, st4 = _sage_combine(acc4, invc, r4, p['sage4']['Wl'],
                            _row(p['sage4']['bl']))

    z4, r5 = _sage_apply(y4, st4, _row(p['bn4']['g']), _row(p['bn4']['b']),
                         p['sage5']['Wr'])
    acc5 = _sage_agg(z4, spk)
    y5, st5 = _sage_combine(acc5, invc, r5, p['sage5']['Wl'],
                            _row(p['sage5']['bl']))

    return _final(y5, st5, _row(p['bn5']['g']), _row(p['bn5']['b']),
                  batf, p['fc']['W'], _row(p['fc']['b']))


# speedup vs baseline: 1.0182x; 1.0052x over previous
import jax
import jax.numpy as jnp
from jax import lax
from jax.experimental import pallas as pl
from jax.experimental.pallas import tpu as pltpu
from jax.experimental.pallas import tpu_sc as plsc

N = 10000
E = 320000
D = 128
HEADS = 4
CH = 32
NG = 16
DOUT = 64

NC, NS, LN = 2, 16, 16
NW = NC * NS
NPAD = 10240
KE = 128

EG = E + N
KEG = 64
CPW_G = 164
CPG0, CPG1 = 196, 132
EPW_G = CPW_G * KEG
EGPAD = NW * EPW_G
CPW_S = 80
CPS0, CPS1 = 120, 40
EPW_S = CPW_S * KE
ESPAD = NW * EPW_S

RB = 1000
NBLK = N // RB

_mesh = plsc.VectorSubcoreMesh(core_axis_name="c", subcore_axis_name="s",
                               num_cores=NC, num_subcores=NS)


def _f32(*shape):
    return jax.ShapeDtypeStruct(shape, jnp.float32)


def _zero_and_dump(acc, zbuf, out_ref, cid, sid, drow, phase):
    base = sid * 640
    if phase == 0:
        def zb(r, _):
            def zc(c, _):
                zbuf[r, pl.ds(c * LN, LN)] = jnp.zeros((LN,), jnp.float32)
                return 0
            lax.fori_loop(0, drow // LN, zc, 0)
            return 0
        lax.fori_loop(0, LN, zb, 0)
        def zrow(j, _):
            pltpu.sync_copy(zbuf, acc.at[pl.ds(base + j * LN, LN)])
            return 0
        lax.fori_loop(0, 640 // LN, zrow, 0)
    else:
        pltpu.sync_copy(acc.at[pl.ds(base, 640)],
                        out_ref.at[cid, pl.ds(base, 640)])


_dnums = lax.GatherDimensionNumbers(
    offset_dims=(), collapsed_slice_dims=(0,), start_index_map=(0,))


def _reg_gather(v, idx):
    return lax.gather(v, idx[:, None], _dnums, (1,),
                      mode=lax.GatherScatterMode.PROMISE_IN_BOUNDS)


def _fetch_idx(pk_hbm, base, ke, pk_v, src_v, dst_v):
    pltpu.sync_copy(pk_hbm.at[pl.ds(base, ke)], pk_v)

    def seg(i, _):
        pk = pk_v[pl.ds(i * LN, LN)]
        src_v[pl.ds(i * LN, LN)] = pk >> 14
        dst_v[pl.ds(i * LN, LN)] = pk & 16383
        return 0
    lax.fori_loop(0, ke // LN, seg, 0)


def _gat_sc_body(t_hbm, at_hbm, gc_hbm, pk_hbm,
                 outf_hbm, outp_hbm,
                 pk0, pk1, srd0, srd1, dsd0, dsd1, in0, in1,
                 as0, as1, ad0, ad1, pc0, pc1, zbuf, zbuf2, gc_v, acc, accp,
                 sg0, sg1, sf0, sf1, sp0, sp1):
    cid = lax.axis_index("c")
    sid = lax.axis_index("s")
    npair = jnp.where(cid == 0, CPG0 // 2, CPG1 // 2)

    _zero_and_dump(acc, zbuf, outf_hbm, cid, sid, D, 0)
    _zero_and_dump(accp, zbuf2, outp_hbm, cid, sid, LN, 0)
    pltpu.sync_copy(gc_hbm, gc_v)
    gc = gc_v[...]

    iot = jax.lax.broadcasted_iota(jnp.int32, (LN,), 0)
    shift4 = (iot & 3) + 4
    one16 = jnp.full((LN,), 1.0, jnp.float32)
    zero16 = jnp.zeros((LN,), jnp.float32)

    plsc.subcore_barrier()

    pks = (pk0, pk1)
    srs = (srd0, srd1)
    dss = (dsd0, dsd1)
    ins = (in0, in1)
    ass = (as0, as1)
    ads = (ad0, ad1)
    pcs = (pc0, pc1)
    sgs = (sg0, sg1)
    sfs = (sf0, sf1)
    sps = (sp0, sp1)

    def g_start(c, b):
        pltpu.async_copy(t_hbm.at[srs[b]], ins[b], sgs[b])
        pltpu.async_copy(at_hbm.at[srs[b]], ass[b], sgs[b])
        pltpu.async_copy(at_hbm.at[dss[b]], ads[b], sgs[b])

    def g_wait(c, b):
        pltpu.make_async_copy(t_hbm.at[srs[b]], ins[b], sgs[b]).wait()
        pltpu.make_async_copy(at_hbm.at[srs[b]], ass[b], sgs[b]).wait()
        pltpu.make_async_copy(at_hbm.at[dss[b]], ads[b], sgs[b]).wait()

    def s_start(c, b):
        pltpu.async_copy(ins[b], acc.at[dss[b]], sfs[b], add=True)
        pltpu.async_copy(pcs[b], accp.at[dss[b]], sps[b], add=True)

    def s_wait(c, b):
        pltpu.make_async_copy(ins[b], acc.at[dss[b]], sfs[b]).wait()
        pltpu.make_async_copy(pcs[b], accp.at[dss[b]], sps[b]).wait()

    def compute(b):
        in_buf = ins[b]
        as_buf = ass[b]
        ad_buf = ads[b]
        pc_buf = pcs[b]

        def row(r, _):
            a_s = as_buf[r, ...]
            a_d = _reg_gather(ad_buf[r, ...], shift4)
            asum = a_s + a_d
            alpha = jnp.where(asum > 0, asum, 0.2 * asum)
            gsum = gc + a_d
            m = jnp.where(gsum > 0, gsum, 0.2 * gsum)
            p = jnp.exp(alpha - m)
            pc_buf[r, ...] = jnp.where(
                iot < 4, p, jnp.where(iot == 4, one16, zero16))
            for h in range(HEADS):
                ph = _reg_gather(p, jnp.full((LN,), h, jnp.int32))
                for j in range(CH // LN):
                    cc = CH * h + LN * j
                    in_buf[r, pl.ds(cc, LN)] = in_buf[r, pl.ds(cc, LN)] * ph
            return 0
        lax.fori_loop(0, KEG, row, 0)

    ebase = sid * (2 * EPW_G) + cid * (CPG0 * KEG)
    _fetch_idx(pk_hbm, ebase, KEG, pk0, srd0, dsd0)
    g_start(0, 0)

    def pair(q, _):
        c0 = q * 2
        c1 = c0 + 1

        @pl.when(q > 0)
        def _():
            s_wait(c1 - 2, 1)
        _fetch_idx(pk_hbm, ebase + c1 * KEG, KEG, pk1, srd1, dsd1)
        g_start(c1, 1)
        g_wait(c0, 0)
        compute(0)
        s_start(c0, 0)

        s_wait(c0, 0)

        @pl.when(q < npair - 1)
        def _():
            _fetch_idx(pk_hbm, ebase + (c0 + 2) * KEG, KEG, pk0, srd0, dsd0)
            g_start(c0 + 2, 0)
        g_wait(c1, 1)
        compute(1)
        s_start(c1, 1)
        return 0
    lax.fori_loop(0, npair, pair, 0)
    s_wait(0, 1)

    plsc.subcore_barrier()
    _zero_and_dump(acc, zbuf, outf_hbm, cid, sid, D, 1)
    _zero_and_dump(accp, zbuf2, outp_hbm, cid, sid, LN, 1)


_sc_params = pltpu.CompilerParams(use_tc_tiling_on_sc=False,
                                  needs_layout_passes=False)

_gat_agg = pl.kernel(
    _gat_sc_body,
    out_type=(_f32(NC, NPAD, D), _f32(NC, NPAD, LN)),
    mesh=_mesh,
    compiler_params=_sc_params,
    scratch_types=[
        pltpu.VMEM((KEG,), jnp.int32),
        pltpu.VMEM((KEG,), jnp.int32),
        pltpu.VMEM((KEG,), jnp.int32),
        pltpu.VMEM((KEG,), jnp.int32),
        pltpu.VMEM((KEG,), jnp.int32),
        pltpu.VMEM((KEG,), jnp.int32),
        pltpu.VMEM((KEG, D), jnp.float32),
        pltpu.VMEM((KEG, D), jnp.float32),
        pltpu.VMEM((KEG, LN), jnp.float32),
        pltpu.VMEM((KEG, LN), jnp.float32),
        pltpu.VMEM((KEG, LN), jnp.float32),
        pltpu.VMEM((KEG, LN), jnp.float32),
        pltpu.VMEM((KEG, LN), jnp.float32),
        pltpu.VMEM((KEG, LN), jnp.float32),
        pltpu.VMEM((LN, D), jnp.float32),
        pltpu.VMEM((LN, LN), jnp.float32),
        pltpu.VMEM((LN,), jnp.float32),
        pltpu.VMEM_SHARED((NPAD, D), jnp.float32),
        pltpu.VMEM_SHARED((NPAD, LN), jnp.float32),
        pltpu.SemaphoreType.DMA,
        pltpu.SemaphoreType.DMA,
        pltpu.SemaphoreType.DMA,
        pltpu.SemaphoreType.DMA,
        pltpu.SemaphoreType.DMA,
        pltpu.SemaphoreType.DMA,
    ],
)


def _sage_sc_body(t_hbm, pk_hbm, out_hbm,
                  pk0, pk1, srd0, srd1, dsd0, dsd1, in0, in1, zbuf, acc,
                  sg0, sg1, ss0, ss1):
    cid = lax.axis_index("c")
    sid = lax.axis_index("s")
    npair = jnp.where(cid == 0, CPS0 // 2, CPS1 // 2)

    _zero_and_dump(acc, zbuf, out_hbm, cid, sid, D, 0)
    plsc.subcore_barrier()

    pks = (pk0, pk1)
    srs = (srd0, srd1)
    dss = (dsd0, dsd1)
    ins = (in0, in1)
    sgs = (sg0, sg1)
    sss = (ss0, ss1)

    def g_start(b):
        pltpu.async_copy(t_hbm.at[srs[b]], ins[b], sgs[b])

    def g_wait(b):
        pltpu.make_async_copy(t_hbm.at[srs[b]], ins[b], sgs[b]).wait()

    def s_start(b):
        pltpu.async_copy(ins[b], acc.at[dss[b]], sss[b], add=True)

    def s_wait(b):
        pltpu.make_async_copy(ins[b], acc.at[dss[b]], sss[b]).wait()

    ebase = sid * (2 * EPW_S) + cid * (CPS0 * KE)
    _fetch_idx(pk_hbm, ebase, KE, pk0, srd0, dsd0)
    g_start(0)

    def pair(q, _):
        c0 = q * 2
        c1 = c0 + 1

        @pl.when(q > 0)
        def _():
            s_wait(1)
        _fetch_idx(pk_hbm, ebase + c1 * KE, KE, pk1, srd1, dsd1)
        g_start(1)
        g_wait(0)
        s_start(0)

        s_wait(0)

        @pl.when(q < npair - 1)
        def _():
            _fetch_idx(pk_hbm, ebase + (c0 + 2) * KE, KE, pk0, srd0, dsd0)
            g_start(0)
        g_wait(1)
        s_start(1)
        return 0
    lax.fori_loop(0, npair, pair, 0)
    s_wait(1)

    plsc.subcore_barrier()
    _zero_and_dump(acc, zbuf, out_hbm, cid, sid, D, 1)


_sage_agg = pl.kernel(
    _sage_sc_body,
    out_type=_f32(NC, NPAD, D),
    mesh=_mesh,
    compiler_params=_sc_params,
    scratch_types=[
        pltpu.VMEM((KE,), jnp.int32),
        pltpu.VMEM((KE,), jnp.int32),
        pltpu.VMEM((KE,), jnp.int32),
        pltpu.VMEM((KE,), jnp.int32),
        pltpu.VMEM((KE,), jnp.int32),
        pltpu.VMEM((KE,), jnp.int32),
        pltpu.VMEM((KE, D), jnp.float32),
        pltpu.VMEM((KE, D), jnp.float32),
        pltpu.VMEM((LN, D), jnp.float32),
        pltpu.VMEM_SHARED((NPAD, D), jnp.float32),
        pltpu.SemaphoreType.DMA,
        pltpu.SemaphoreType.DMA,
        pltpu.SemaphoreType.DMA,
        pltpu.SemaphoreType.DMA,
    ],
)


_rowspec = pl.BlockSpec((RB, D), lambda i: (i, 0))
_fullw = pl.BlockSpec((D, D), lambda i: (0, 0))
_statspec = pl.BlockSpec((8, D), lambda i: (0, 0))


def _dot(a, b):
    return jax.lax.dot_general(a, b, (((1,), (0,)), ((), ())),
                               preferred_element_type=jnp.float32)


def _lrelu(v):
    return jnp.where(v > 0, v, 0.2 * v)


def _bn_apply(y, st_ref, g_ref, b_ref):
    mu = st_ref[0:1, :] / N
    var = st_ref[1:2, :] / N - mu * mu
    inv = jax.lax.rsqrt(var + 1e-5)
    return (y - mu) * inv * g_ref[0:1, :] + b_ref[0:1, :]


def _gat_prep_out(h, as_ref, ad_ref, gm_ref, t_ref, att_ref, i):
    asrc = _dot(h, as_ref[...])
    adst = _dot(h, ad_ref[...])
    t_ref[...] = h
    att_ref[...] = jnp.concatenate(
        [asrc, adst, jnp.zeros((RB, LN - 2 * HEADS), jnp.float32)], axis=1)
    prev = jnp.where(i == 0, jnp.full((1, HEADS), -1e30, jnp.float32),
                     gm_ref[0:1, 0:HEADS])
    gm_ref[0:1, 0:HEADS] = jnp.maximum(prev, jnp.max(asrc, axis=0, keepdims=True))


def _stage_a_body(x_ref, w_ref, as_ref, ad_ref, wr_ref, br_ref,
                  t_ref, adt_ref, res_ref, gm_ref):
    i = pl.program_id(0)
    xb = x_ref[...]
    h = _dot(xb, w_ref[...])
    res_ref[...] = _dot(xb, wr_ref[...]) + br_ref[0:1, :]
    _gat_prep_out(h, as_ref, ad_ref, gm_ref, t_ref, adt_ref, i)


_stage_a = pl.pallas_call(
    _stage_a_body,
    grid=(NBLK,),
    in_specs=[_rowspec, _fullw,
              pl.BlockSpec((D, HEADS), lambda i: (0, 0)),
              pl.BlockSpec((D, HEADS), lambda i: (0, 0)),
              _fullw, pl.BlockSpec((1, D), lambda i: (0, 0))],
    out_specs=[_rowspec,
               pl.BlockSpec((RB, LN), lambda i: (i, 0)),
               _rowspec, _statspec],
    out_shape=[_f32(N, D), _f32(N, LN), _f32(N, D), _f32(8, D)],
)


def _gat_combine_body(acc_ref, accp_ref, bg_ref, y_ref, st_ref, ic_ref):
    i = pl.program_id(0)
    s = acc_ref[0] + acc_ref[1]
    pc = accp_ref[0] + accp_ref[1]
    den = jnp.concatenate(
        [jnp.broadcast_to(pc[:, h:h + 1], (RB, CH)) for h in range(HEADS)],
        axis=1)
    y = s / (den + 1e-16) + bg_ref[0:1, :]
    y_ref[...] = y
    prev0 = jnp.where(i == 0, jnp.zeros((1, D)), st_ref[0:1, :])
    prev1 = jnp.where(i == 0, jnp.zeros((1, D)), st_ref[1:2, :])
    st_ref[0:1, :] = prev0 + jnp.sum(y, axis=0, keepdims=True)
    st_ref[1:2, :] = prev1 + jnp.sum(y * y, axis=0, keepdims=True)
    ic_ref[...] = 1.0 / jnp.maximum(pc[:, HEADS:HEADS + 1] - 1.0, 1.0)


_gat_combine = pl.pallas_call(
    _gat_combine_body,
    grid=(NBLK,),
    in_specs=[pl.BlockSpec((NC, RB, D), lambda i: (0, i, 0)),
              pl.BlockSpec((NC, RB, LN), lambda i: (0, i, 0)),
              pl.BlockSpec((1, D), lambda i: (0, 0))],
    out_specs=[_rowspec, _statspec, pl.BlockSpec((RB, 1), lambda i: (i, 0))],
    out_shape=[_f32(N, D), _f32(8, D), _f32(N, 1)],
)


def _gat_apply_prep_body(y_ref, st_ref, res_ref, g_ref, b_ref,
                         w_ref, as_ref, ad_ref, wr_ref, br_ref,
                         t_ref, adt_ref, res2_ref, gm_ref):
    i = pl.program_id(0)
    z = _lrelu(_bn_apply(y_ref[...], st_ref, g_ref, b_ref) + res_ref[...])
    h = _dot(z, w_ref[...])
    res2_ref[...] = _dot(z, wr_ref[...]) + br_ref[0:1, :]
    _gat_prep_out(h, as_ref, ad_ref, gm_ref, t_ref, adt_ref, i)


_gat_apply_prep = pl.pallas_call(
    _gat_apply_prep_body,
    grid=(NBLK,),
    in_specs=[_rowspec, _statspec, _rowspec,
              pl.BlockSpec((1, D), lambda i: (0, 0)),
              pl.BlockSpec((1, D), lambda i: (0, 0)),
              _fullw,
              pl.BlockSpec((D, HEADS), lambda i: (0, 0)),
              pl.BlockSpec((D, HEADS), lambda i: (0, 0)),
              _fullw, pl.BlockSpec((1, D), lambda i: (0, 0))],
    out_specs=[_rowspec,
               pl.BlockSpec((RB, LN), lambda i: (i, 0)),
               _rowspec, _statspec],
    out_shape=[_f32(N, D), _f32(N, LN), _f32(N, D), _f32(8, D)],
)


def _gat_apply_sage_body(y_ref, st_ref, res_ref, g_ref, b_ref, wr_ref,
                         z_ref, r_ref):
    z = _lrelu(_bn_apply(y_ref[...], st_ref, g_ref, b_ref) + res_ref[...])
    z_ref[...] = z
    r_ref[...] = _dot(z, wr_ref[...])


_gat_apply_sage = pl.pallas_call(
    _gat_apply_sage_body,
    grid=(NBLK,),
    in_specs=[_rowspec, _statspec, _rowspec,
              pl.BlockSpec((1, D), lambda i: (0, 0)),
              pl.BlockSpec((1, D), lambda i: (0, 0)),
              _fullw],
    out_specs=[_rowspec, _rowspec],
    out_shape=[_f32(N, D), _f32(N, D)],
)


def _sage_combine_body(acc_ref, ic_ref, r_ref, wl_ref, bl_ref, y_ref, st_ref):
    i = pl.program_id(0)
    s = acc_ref[0] + acc_ref[1]
    mean = s * ic_ref[...]
    y = _dot(mean, wl_ref[...]) + bl_ref[0:1, :] + r_ref[...]
    y_ref[...] = y
    prev0 = jnp.where(i == 0, jnp.zeros((1, D)), st_ref[0:1, :])
    prev1 = jnp.where(i == 0, jnp.zeros((1, D)), st_ref[1:2, :])
    st_ref[0:1, :] = prev0 + jnp.sum(y, axis=0, keepdims=True)
    st_ref[1:2, :] = prev1 + jnp.sum(y * y, axis=0, keepdims=True)


_sage_combine = pl.pallas_call(
    _sage_combine_body,
    grid=(NBLK,),
    in_specs=[pl.BlockSpec((NC, RB, D), lambda i: (0, i, 0)),
              pl.BlockSpec((RB, 1), lambda i: (i, 0)),
              _rowspec, _fullw, pl.BlockSpec((1, D), lambda i: (0, 0))],
    out_specs=[_rowspec, _statspec],
    out_shape=[_f32(N, D), _f32(8, D)],
)


def _sage_apply_body(y_ref, st_ref, g_ref, b_ref, wr_ref, z_ref, r_ref):
    z = _lrelu(_bn_apply(y_ref[...], st_ref, g_ref, b_ref))
    z_ref[...] = z
    r_ref[...] = _dot(z, wr_ref[...])


_sage_apply = pl.pallas_call(
    _sage_apply_body,
    grid=(NBLK,),
    in_specs=[_rowspec, _statspec,
              pl.BlockSpec((1, D), lambda i: (0, 0)),
              pl.BlockSpec((1, D), lambda i: (0, 0)),
              _fullw],
    out_specs=[_rowspec, _rowspec],
    out_shape=[_f32(N, D), _f32(N, D)],
)


def _final_body(y_ref, st_ref, g_ref, b_ref, bt_ref, wf_ref, bf_ref,
                out_ref, gacc):
    i = pl.program_id(0)
    z = _lrelu(_bn_apply(y_ref[...], st_ref, g_ref, b_ref))
    zext = jnp.concatenate([z, jnp.ones((RB, 8), jnp.float32)], axis=1)
    gid = jax.lax.broadcasted_iota(jnp.int32, (NG, 1), 0).astype(jnp.float32)
    oh = jnp.where(gid == bt_ref[0], 1.0, 0.0)
    prev = jnp.where(i == 0, jnp.zeros((NG, D + 8)), gacc[...])
    gsum = prev + _dot(oh, zext)
    gacc[...] = gsum

    @pl.when(i == NBLK - 1)
    def _():
        gm = gsum[:, :D] / jnp.maximum(gsum[:, D:D + 1], 1.0)
        out_ref[...] = _dot(gm, wf_ref[...]) + bf_ref[0:1, :]


_final = pl.pallas_call(
    _final_body,
    grid=(NBLK,),
    in_specs=[_rowspec, _statspec,
              pl.BlockSpec((1, D), lambda i: (0, 0)),
              pl.BlockSpec((1, D), lambda i: (0, 0)),
              pl.BlockSpec((1, 1, RB), lambda i: (i, 0, 0)),
              pl.BlockSpec((D, DOUT), lambda i: (0, 0)),
              pl.BlockSpec((1, DOUT), lambda i: (0, 0))],
    out_specs=pl.BlockSpec((NG, DOUT), lambda i: (0, 0)),
    out_shape=_f32(NG, DOUT),
    scratch_shapes=[pltpu.VMEM((NG, D + 8), jnp.float32)],
)


def _att_mat(att):
    m = jnp.zeros((D, HEADS), jnp.float32)
    for h in range(HEADS):
        m = m.at[h * CH:(h + 1) * CH, h].set(att[0, h])
    return m


def _row(v):
    return v.reshape(1, -1)


def kernel(x, edge_index, batch, params):
    p = params
    loop = jnp.arange(N, dtype=jnp.int32)
    padg = EGPAD - EG
    gpk = jnp.concatenate([
        (edge_index[0] << 14) | edge_index[1],
        (loop << 14) | loop,
        jnp.full((padg,), N, jnp.int32)])
    pads = ESPAD - E
    spk = jnp.concatenate([
        (edge_index[0] << 14) | edge_index[1],
        jnp.full((pads,), N, jnp.int32)])
    batf = batch.astype(jnp.float32).reshape(NBLK, 1, RB)

    g1 = p['gat1']
    t1, att1, res1, gm1 = _stage_a(
        x, g1['W'], _att_mat(g1['att_src']), _att_mat(g1['att_dst']),
        p['res1']['W'], _row(p['res1']['b']))
    gc1 = jnp.tile(gm1[0, :HEADS], HEADS)
    att1 = jnp.concatenate([att1, jnp.zeros((LN, LN), jnp.float32)])
    acc1, accp1 = _gat_agg(t1, att1, gc1, gpk)
    y1, st1, invc = _gat_combine(acc1, accp1, _row(g1['b']))

    g2 = p['gat2']
    t2, att2, res2, gm2 = _gat_apply_prep(
        y1, st1, res1, _row(p['bn1']['g']), _row(p['bn1']['b']),
        g2['W'], _att_mat(g2['att_src']), _att_mat(g2['att_dst']),
        p['res2']['W'], _row(p['res2']['b']))
    gc2 = jnp.tile(gm2[0, :HEADS], HEADS)
    att2 = jnp.concatenate([att2, jnp.zeros((LN, LN), jnp.float32)])
    acc2, accp2 = _gat_agg(t2, att2, gc2, gpk)
    y2, st2, _ = _gat_combine(acc2, accp2, _row(g2['b']))

    z2, r3 = _gat_apply_sage(y2, st2, res2, _row(p['bn2']['g']),
                             _row(p['bn2']['b']), p['sage3']['Wr'])
    acc3 = _sage_agg(z2, spk)
    y3, st3 = _sage_combine(acc3, invc, r3, p['sage3']['Wl'],
                            _row(p['sage3']['bl']))

    z3, r4 = _sage_apply(y3, st3, _row(p['bn3']['g']), _row(p['bn3']['b']),
                         p['sage4']['Wr'])
    acc4 = _sage_agg(z3, spk)
    y4, st4 = _sage_combine(acc4, invc, r4, p['sage4']['Wl'],
                            _row(p['sage4']['bl']))

    z4, r5 = _sage_apply(y4, st4, _row(p['bn4']['g']), _row(p['bn4']['b']),
                         p['sage5']['Wr'])
    acc5 = _sage_agg(z4, spk)
    y5, st5 = _sage_combine(acc5, invc, r5, p['sage5']['Wl'],
                            _row(p['sage5']['bl']))

    return _final(y5, st5, _row(p['bn5']['g']), _row(p['bn5']['b']),
                  batf, p['fc']['W'], _row(p['fc']['b']))
